# baseline (speedup 1.0000x reference)
import functools

import jax
import jax.numpy as jnp
from jax import lax
from jax.experimental import pallas as pl
from jax.experimental.pallas import tpu as pltpu
from jax.experimental.pallas import tpu_sc as plsc

VOCAB = 1000000
EMBED = 64
BATCH = 16384
POS = 10
NEG = 50

NC = 2
NS = 16
NW = NC * NS

CH = 512

N_IN = BATCH
N_POS = BATCH * POS
N_NEG = BATCH * NEG

TL = 512
BL = 2048


def _tt_body(x_ref, o_ref):
    o_ref[...] = x_ref[...].T


def _tc_table_transpose(tab_t):
    grid = (pl.cdiv(VOCAB, TL),)
    return pl.pallas_call(
        _tt_body,
        grid=grid,
        in_specs=[pl.BlockSpec((EMBED, TL), lambda i: (0, i))],
        out_specs=pl.BlockSpec((TL, EMBED), lambda i: (i, 0)),
        out_shape=jax.ShapeDtypeStruct((VOCAB, EMBED), jnp.float32),
    )(tab_t)


def _to3_body(x_ref, o_ref):
    o_ref[0, :, :] = x_ref[0, :, :].T


def _tc_out_transpose3(g, k):
    grid = (k, BATCH // BL)
    return pl.pallas_call(
        _to3_body,
        grid=grid,
        in_specs=[pl.BlockSpec((1, BL, EMBED), lambda i, j: (i, j, 0))],
        out_specs=pl.BlockSpec((1, EMBED, BL), lambda i, j: (i, 0, j)),
        out_shape=jax.ShapeDtypeStruct((k, EMBED, BATCH), jnp.float32),
    )(g)


def _to2_body(x_ref, o_ref):
    o_ref[...] = x_ref[...].T


def _tc_out_transpose2(g):
    grid = (BATCH // BL,)
    return pl.pallas_call(
        _to2_body,
        grid=grid,
        in_specs=[pl.BlockSpec((BL, EMBED), lambda i: (i, 0))],
        out_specs=pl.BlockSpec((EMBED, BL), lambda i: (0, i)),
        out_shape=jax.ShapeDtypeStruct((EMBED, BATCH), jnp.float32),
    )(g)


def _gather_desc(table, idx_vmem, rows_v, sem, c):
    return pltpu.make_async_copy(
        table.at[idx_vmem.at[pl.ds(c * CH, CH)]], rows_v, sem)


def _store_desc(out_hbm, rows_v, sem, base, c):
    return pltpu.make_async_copy(
        rows_v, out_hbm.at[pl.ds(base + c * CH, CH)], sem)


def _gather_range(table, idx_vmem, out_hbm, rows_a, rows_b,
                  gsa, gsb, ssa, ssb, base, nch):
    if nch == 1:
        _gather_desc(table, idx_vmem, rows_a, gsa, 0).start()
        _gather_desc(table, idx_vmem, rows_a, gsa, 0).wait()
        pltpu.sync_copy(rows_a, out_hbm.at[pl.ds(base, CH)])
        return

    _gather_desc(table, idx_vmem, rows_a, gsa, 0).start()

    def pair(p, carry):
        g = 2 * p
        _gather_desc(table, idx_vmem, rows_b, gsb, g + 1).start()
        _gather_desc(table, idx_vmem, rows_a, gsa, g).wait()
        _store_desc(out_hbm, rows_a, ssa, base, g).start()

        @pl.when(g + 2 < nch)
        def _():
            _store_desc(out_hbm, rows_a, ssa, base, g).wait()
            _gather_desc(table, idx_vmem, rows_a, gsa, g + 2).start()

        _gather_desc(table, idx_vmem, rows_b, gsb, g + 1).wait()
        _store_desc(out_hbm, rows_b, ssb, base, g + 1).start()
        _store_desc(out_hbm, rows_b, ssb, base, g + 1).wait()
        return carry

    lax.fori_loop(0, nch // 2, pair, None)
    _store_desc(out_hbm, rows_a, ssa, base, nch - 2).wait()


def _body(in_table, out_table, in_lbl, pos_lbl, neg_lbl,
          out_in, out_pos, out_neg,
          idx_in_v, idx_pos_v, idx_neg_v, rows_a, rows_b,
          gsa, gsb, ssa, ssb):
    wid = lax.axis_index("s") * NC + lax.axis_index("c")

    in_pw = N_IN // NW
    pos_pw = N_POS // NW
    neg_pw = N_NEG // NW

    pltpu.sync_copy(in_lbl.at[pl.ds(wid * in_pw, in_pw)], idx_in_v)
    pltpu.sync_copy(pos_lbl.at[pl.ds(wid * pos_pw, pos_pw)], idx_pos_v)
    pltpu.sync_copy(neg_lbl.at[pl.ds(wid * neg_pw, neg_pw)], idx_neg_v)

    _gather_range(in_table, idx_in_v, out_in, rows_a, rows_b,
                  gsa, gsb, ssa, ssb, wid * in_pw, in_pw // CH)
    _gather_range(out_table, idx_pos_v, out_pos, rows_a, rows_b,
                  gsa, gsb, ssa, ssb, wid * pos_pw, pos_pw // CH)
    _gather_range(out_table, idx_neg_v, out_neg, rows_a, rows_b,
                  gsa, gsb, ssa, ssb, wid * neg_pw, neg_pw // CH)


def _sc_gather(in_table, out_table, in_lbl, pos_lbl, neg_lbl):
    mesh = plsc.VectorSubcoreMesh(core_axis_name="c", subcore_axis_name="s")
    f = pl.kernel(
        _body,
        out_type=[
            jax.ShapeDtypeStruct((N_IN, EMBED), jnp.float32),
            jax.ShapeDtypeStruct((N_POS, EMBED), jnp.float32),
            jax.ShapeDtypeStruct((N_NEG, EMBED), jnp.float32),
        ],
        mesh=mesh,
        compiler_params=pltpu.CompilerParams(use_tc_tiling_on_sc=False),
        scratch_types=[
            pltpu.VMEM((N_IN // NW,), jnp.int32),
            pltpu.VMEM((N_POS // NW,), jnp.int32),
            pltpu.VMEM((N_NEG // NW,), jnp.int32),
            pltpu.VMEM((CH, EMBED), jnp.float32),
            pltpu.VMEM((CH, EMBED), jnp.float32),
            pltpu.SemaphoreType.DMA,
            pltpu.SemaphoreType.DMA,
            pltpu.SemaphoreType.DMA,
            pltpu.SemaphoreType.DMA,
        ],
    )
    return f(in_table, out_table, in_lbl, pos_lbl, neg_lbl)


@jax.jit
def _run(input_labels, pos_labels, neg_labels, in_table, out_table):
    in_rm = _tc_table_transpose(in_table.T)
    out_rm = _tc_table_transpose(out_table.T)

    in_lbl = input_labels.astype(jnp.int32)
    pos_flat = pos_labels.astype(jnp.int32).reshape(-1)
    neg_flat = neg_labels.astype(jnp.int32).reshape(-1)

    g_in, g_pos, g_neg = _sc_gather(in_rm, out_rm, in_lbl, pos_flat, neg_flat)

    return (g_in,
            g_pos.reshape(BATCH, POS, EMBED),
            g_neg.reshape(BATCH, NEG, EMBED))


def kernel(input_labels, pos_labels, neg_labels, in_table, out_table):
    return _run(input_labels, pos_labels, neg_labels, in_table, out_table)


# speedup vs baseline: 1.1972x; 1.1972x over previous
import functools

import jax
import jax.numpy as jnp
from jax import lax
from jax.experimental import pallas as pl
from jax.experimental.pallas import tpu as pltpu
from jax.experimental.pallas import tpu_sc as plsc

VOCAB = 1000000
EMBED = 64
BATCH = 16384
POS = 10
NEG = 50

NC = 2
NS = 16
NW = NC * NS

CH = 512

N_IN = BATCH
N_POS = BATCH * POS
N_NEG = BATCH * NEG

TL = 512
BL = 2048

NTB = (VOCAB + TL - 1) // TL
VPAD = NTB * TL


def _tt_body(x_ref, o_ref):
    x = x_ref[...]
    a = x[:, : TL // 2].T
    b = x[:, TL // 2:].T
    o_ref[...] = jnp.concatenate([a, b], axis=1)


def _tc_table_transpose(tab_t):
    return pl.pallas_call(
        _tt_body,
        grid=(NTB,),
        in_specs=[pl.BlockSpec((EMBED, TL), lambda i: (0, i))],
        out_specs=pl.BlockSpec((TL // 2, 2 * EMBED), lambda i: (i, 0)),
        out_shape=jax.ShapeDtypeStruct((VPAD // 2, 2 * EMBED), jnp.float32),
    )(tab_t)


def _to_body(x_ref, o_ref):
    x = x_ref[...]
    a = x[:, :EMBED].T
    b = x[:, EMBED:].T
    o_ref[0, :, :] = jnp.concatenate([a, b], axis=1)


def _tc_out_transpose3(g2, k):
    nb = BATCH // BL
    return pl.pallas_call(
        _to_body,
        grid=(k, nb),
        in_specs=[pl.BlockSpec((BL // 2, 2 * EMBED),
                               lambda i, j: (i * nb + j, 0))],
        out_specs=pl.BlockSpec((1, EMBED, BL), lambda i, j: (i, 0, j)),
        out_shape=jax.ShapeDtypeStruct((k, EMBED, BATCH), jnp.float32),
    )(g2)


def _to2_body(x_ref, o_ref):
    x = x_ref[...]
    a = x[:, :EMBED].T
    b = x[:, EMBED:].T
    o_ref[...] = jnp.concatenate([a, b], axis=1)


def _tc_out_transpose2(g2):
    nb = BATCH // BL
    return pl.pallas_call(
        _to2_body,
        grid=(nb,),
        in_specs=[pl.BlockSpec((BL // 2, 2 * EMBED), lambda j: (j, 0))],
        out_specs=pl.BlockSpec((EMBED, BL), lambda j: (0, j)),
        out_shape=jax.ShapeDtypeStruct((EMBED, BATCH), jnp.float32),
    )(g2)


def _pi(r):
    blk = (r // TL) * TL
    q = r % TL
    return blk + 2 * (q % (TL // 2)) + q // (TL // 2)


def _sigma(lbl_k_major, k):
    x = lbl_k_major.reshape(k, BATCH // BL, 2, BL // 2)
    return x.swapaxes(2, 3).reshape(-1)


def _gather_desc(table, idx_vmem, rows_v, sem, c):
    return pltpu.make_async_copy(
        table.at[idx_vmem.at[pl.ds(c * CH, CH)]], rows_v, sem)


def _store_desc(out_hbm, rows_v, sem, base, c):
    return pltpu.make_async_copy(
        rows_v, out_hbm.at[pl.ds(base + c * CH, CH)], sem)


def _gather_range(table, idx_vmem, out_hbm, rows_a, rows_b,
                  gsa, gsb, ssa, ssb, base, nch):
    if nch == 1:
        _gather_desc(table, idx_vmem, rows_a, gsa, 0).start()
        _gather_desc(table, idx_vmem, rows_a, gsa, 0).wait()
        pltpu.sync_copy(rows_a, out_hbm.at[pl.ds(base, CH)])
        return

    _gather_desc(table, idx_vmem, rows_a, gsa, 0).start()

    def pair(p, carry):
        g = 2 * p
        _gather_desc(table, idx_vmem, rows_b, gsb, g + 1).start()
        _gather_desc(table, idx_vmem, rows_a, gsa, g).wait()
        _store_desc(out_hbm, rows_a, ssa, base, g).start()

        @pl.when(g + 2 < nch)
        def _():
            _store_desc(out_hbm, rows_a, ssa, base, g).wait()
            _gather_desc(table, idx_vmem, rows_a, gsa, g + 2).start()

        _gather_desc(table, idx_vmem, rows_b, gsb, g + 1).wait()
        _store_desc(out_hbm, rows_b, ssb, base, g + 1).start()
        _store_desc(out_hbm, rows_b, ssb, base, g + 1).wait()
        return carry

    lax.fori_loop(0, nch // 2, pair, None)
    _store_desc(out_hbm, rows_a, ssa, base, nch - 2).wait()


def _body(in_table, out_table, in_lbl, pos_lbl, neg_lbl,
          out_in, out_pos, out_neg,
          idx_in_v, idx_pos_v, idx_neg_v, rows_a, rows_b,
          gsa, gsb, ssa, ssb):
    wid = lax.axis_index("s") * NC + lax.axis_index("c")

    in_pw = N_IN // NW
    pos_pw = N_POS // NW
    neg_pw = N_NEG // NW

    pltpu.sync_copy(in_lbl.at[pl.ds(wid * in_pw, in_pw)], idx_in_v)
    pltpu.sync_copy(pos_lbl.at[pl.ds(wid * pos_pw, pos_pw)], idx_pos_v)
    pltpu.sync_copy(neg_lbl.at[pl.ds(wid * neg_pw, neg_pw)], idx_neg_v)

    _gather_range(in_table, idx_in_v, out_in, rows_a, rows_b,
                  gsa, gsb, ssa, ssb, wid * in_pw, in_pw // CH)
    _gather_range(out_table, idx_pos_v, out_pos, rows_a, rows_b,
                  gsa, gsb, ssa, ssb, wid * pos_pw, pos_pw // CH)
    _gather_range(out_table, idx_neg_v, out_neg, rows_a, rows_b,
                  gsa, gsb, ssa, ssb, wid * neg_pw, neg_pw // CH)


def _sc_gather(in_table, out_table, in_lbl, pos_lbl, neg_lbl):
    mesh = plsc.VectorSubcoreMesh(core_axis_name="c", subcore_axis_name="s")
    f = pl.kernel(
        _body,
        out_type=[
            jax.ShapeDtypeStruct((N_IN, EMBED), jnp.float32),
            jax.ShapeDtypeStruct((N_POS, EMBED), jnp.float32),
            jax.ShapeDtypeStruct((N_NEG, EMBED), jnp.float32),
        ],
        mesh=mesh,
        compiler_params=pltpu.CompilerParams(use_tc_tiling_on_sc=False),
        scratch_types=[
            pltpu.VMEM((N_IN // NW,), jnp.int32),
            pltpu.VMEM((N_POS // NW,), jnp.int32),
            pltpu.VMEM((N_NEG // NW,), jnp.int32),
            pltpu.VMEM((CH, EMBED), jnp.float32),
            pltpu.VMEM((CH, EMBED), jnp.float32),
            pltpu.SemaphoreType.DMA,
            pltpu.SemaphoreType.DMA,
            pltpu.SemaphoreType.DMA,
            pltpu.SemaphoreType.DMA,
        ],
    )
    return f(in_table, out_table, in_lbl, pos_lbl, neg_lbl)


@jax.jit
def _run(input_labels, pos_labels, neg_labels, in_table, out_table):
    in_pairs = _tc_table_transpose(in_table.T)
    out_pairs = _tc_table_transpose(out_table.T)
    in_rm = in_pairs.reshape(VPAD, EMBED)
    out_rm = out_pairs.reshape(VPAD, EMBED)

    in_lbl = _sigma(_pi(input_labels.astype(jnp.int32)), 1)
    pos_lbl = _sigma(_pi(pos_labels.astype(jnp.int32).T), POS)
    neg_lbl = _sigma(_pi(neg_labels.astype(jnp.int32).T), NEG)

    g_in, g_pos, g_neg = _sc_gather(in_rm, out_rm, in_lbl, pos_lbl, neg_lbl)

    o_in = _tc_out_transpose2(g_in.reshape(BATCH // 2, 2 * EMBED))
    o_pos = _tc_out_transpose3(g_pos.reshape(N_POS // 2, 2 * EMBED), POS)
    o_neg = _tc_out_transpose3(g_neg.reshape(N_NEG // 2, 2 * EMBED), NEG)

    return (o_in.T,
            o_pos.transpose(2, 0, 1),
            o_neg.transpose(2, 0, 1))


def kernel(input_labels, pos_labels, neg_labels, in_table, out_table):
    return _run(input_labels, pos_labels, neg_labels, in_table, out_table)


# speedup vs baseline: 2.3479x; 1.9611x over previous
import functools

import jax
import jax.numpy as jnp
from jax import lax
from jax.experimental import pallas as pl
from jax.experimental.pallas import tpu as pltpu
from jax.experimental.pallas import tpu_sc as plsc

VOCAB = 1000000
EMBED = 64
BATCH = 16384
POS = 10
NEG = 50

NC = 2
NS = 16
NW = NC * NS

CH = 512

N_IN = BATCH
N_POS = BATCH * POS
N_NEG = BATCH * NEG

TL = 2048
BL = 4096

NTB = (VOCAB + TL - 1) // TL
VPAD = NTB * TL


def _tt_body(x_ref, o_ref):
    x = x_ref[...]
    a = x[:, : TL // 2].T
    b = x[:, TL // 2:].T
    o_ref[...] = jnp.concatenate([a, b], axis=1)


def _tc_table_transpose(tab_t):
    return pl.pallas_call(
        _tt_body,
        grid=(NTB,),
        in_specs=[pl.BlockSpec((EMBED, TL), lambda i: (0, i))],
        out_specs=pl.BlockSpec((TL // 2, 2 * EMBED), lambda i: (i, 0)),
        out_shape=jax.ShapeDtypeStruct((VPAD // 2, 2 * EMBED), jnp.float32),
    )(tab_t)


def _to_body(x_ref, o_ref):
    x = x_ref[...]
    a = x[:, :EMBED].T
    b = x[:, EMBED:].T
    o_ref[0, :, :] = jnp.concatenate([a, b], axis=1)


def _tc_out_transpose3(g2, k):
    nb = BATCH // BL
    return pl.pallas_call(
        _to_body,
        grid=(k, nb),
        in_specs=[pl.BlockSpec((BL // 2, 2 * EMBED),
                               lambda i, j: (i * nb + j, 0))],
        out_specs=pl.BlockSpec((1, EMBED, BL), lambda i, j: (i, 0, j)),
        out_shape=jax.ShapeDtypeStruct((k, EMBED, BATCH), jnp.float32),
    )(g2)


def _to2_body(x_ref, o_ref):
    x = x_ref[...]
    a = x[:, :EMBED].T
    b = x[:, EMBED:].T
    o_ref[...] = jnp.concatenate([a, b], axis=1)


def _tc_out_transpose2(g2):
    nb = BATCH // BL
    return pl.pallas_call(
        _to2_body,
        grid=(nb,),
        in_specs=[pl.BlockSpec((BL // 2, 2 * EMBED), lambda j: (j, 0))],
        out_specs=pl.BlockSpec((EMBED, BL), lambda j: (0, j)),
        out_shape=jax.ShapeDtypeStruct((EMBED, BATCH), jnp.float32),
    )(g2)


def _pi(r):
    blk = (r // TL) * TL
    q = r % TL
    return blk + 2 * (q % (TL // 2)) + q // (TL // 2)


def _sigma(lbl_k_major, k):
    x = lbl_k_major.reshape(k, BATCH // BL, 2, BL // 2)
    return x.swapaxes(2, 3).reshape(-1)


def _gather_desc(table, idx_vmem, rows_v, sem, c):
    return pltpu.make_async_copy(
        table.at[idx_vmem.at[pl.ds(c * CH, CH)]], rows_v, sem)


def _store_desc(out_hbm, rows_v, sem, base, c):
    return pltpu.make_async_copy(
        rows_v, out_hbm.at[pl.ds(base + c * CH, CH)], sem)


def _gather_range(table, idx_vmem, out_hbm, rows_a, rows_b,
                  gsa, gsb, ssa, ssb, base, nch):
    if nch == 1:
        _gather_desc(table, idx_vmem, rows_a, gsa, 0).start()
        _gather_desc(table, idx_vmem, rows_a, gsa, 0).wait()
        pltpu.sync_copy(rows_a, out_hbm.at[pl.ds(base, CH)])
        return

    _gather_desc(table, idx_vmem, rows_a, gsa, 0).start()

    def pair(p, carry):
        g = 2 * p
        _gather_desc(table, idx_vmem, rows_b, gsb, g + 1).start()
        _gather_desc(table, idx_vmem, rows_a, gsa, g).wait()
        _store_desc(out_hbm, rows_a, ssa, base, g).start()

        @pl.when(g + 2 < nch)
        def _():
            _store_desc(out_hbm, rows_a, ssa, base, g).wait()
            _gather_desc(table, idx_vmem, rows_a, gsa, g + 2).start()

        _gather_desc(table, idx_vmem, rows_b, gsb, g + 1).wait()
        _store_desc(out_hbm, rows_b, ssb, base, g + 1).start()
        _store_desc(out_hbm, rows_b, ssb, base, g + 1).wait()
        return carry

    lax.fori_loop(0, nch // 2, pair, None)
    _store_desc(out_hbm, rows_a, ssa, base, nch - 2).wait()


def _body(in_table, out_table, in_lbl, pos_lbl, neg_lbl,
          out_in, out_pos, out_neg,
          idx_in_v, idx_pos_v, idx_neg_v, rows_a, rows_b,
          gsa, gsb, ssa, ssb):
    wid = lax.axis_index("s") * NC + lax.axis_index("c")

    in_pw = N_IN // NW
    pos_pw = N_POS // NW
    neg_pw = N_NEG // NW

    pltpu.sync_copy(in_lbl.at[pl.ds(wid * in_pw, in_pw)], idx_in_v)
    pltpu.sync_copy(pos_lbl.at[pl.ds(wid * pos_pw, pos_pw)], idx_pos_v)
    pltpu.sync_copy(neg_lbl.at[pl.ds(wid * neg_pw, neg_pw)], idx_neg_v)

    _gather_range(in_table, idx_in_v, out_in, rows_a, rows_b,
                  gsa, gsb, ssa, ssb, wid * in_pw, in_pw // CH)
    _gather_range(out_table, idx_pos_v, out_pos, rows_a, rows_b,
                  gsa, gsb, ssa, ssb, wid * pos_pw, pos_pw // CH)
    _gather_range(out_table, idx_neg_v, out_neg, rows_a, rows_b,
                  gsa, gsb, ssa, ssb, wid * neg_pw, neg_pw // CH)


def _sc_gather(in_table, out_table, in_lbl, pos_lbl, neg_lbl):
    mesh = plsc.VectorSubcoreMesh(core_axis_name="c", subcore_axis_name="s")
    f = pl.kernel(
        _body,
        out_type=[
            jax.ShapeDtypeStruct((N_IN, EMBED), jnp.float32),
            jax.ShapeDtypeStruct((N_POS, EMBED), jnp.float32),
            jax.ShapeDtypeStruct((N_NEG, EMBED), jnp.float32),
        ],
        mesh=mesh,
        compiler_params=pltpu.CompilerParams(use_tc_tiling_on_sc=False),
        scratch_types=[
            pltpu.VMEM((N_IN // NW,), jnp.int32),
            pltpu.VMEM((N_POS // NW,), jnp.int32),
            pltpu.VMEM((N_NEG // NW,), jnp.int32),
            pltpu.VMEM((CH, EMBED), jnp.float32),
            pltpu.VMEM((CH, EMBED), jnp.float32),
            pltpu.SemaphoreType.DMA,
            pltpu.SemaphoreType.DMA,
            pltpu.SemaphoreType.DMA,
            pltpu.SemaphoreType.DMA,
        ],
    )
    return f(in_table, out_table, in_lbl, pos_lbl, neg_lbl)


@jax.jit
def _run(input_labels, pos_labels, neg_labels, in_table, out_table):
    in_pairs = _tc_table_transpose(in_table.T)
    out_pairs = _tc_table_transpose(out_table.T)
    in_rm = in_pairs.reshape(VPAD, EMBED)
    out_rm = out_pairs.reshape(VPAD, EMBED)

    in_lbl = _sigma(_pi(input_labels.astype(jnp.int32)), 1)
    pos_lbl = _sigma(_pi(pos_labels.astype(jnp.int32).T), POS)
    neg_lbl = _sigma(_pi(neg_labels.astype(jnp.int32).T), NEG)

    g_in, g_pos, g_neg = _sc_gather(in_rm, out_rm, in_lbl, pos_lbl, neg_lbl)

    o_in = _tc_out_transpose2(g_in.reshape(BATCH // 2, 2 * EMBED))
    o_pos = _tc_out_transpose3(g_pos.reshape(N_POS // 2, 2 * EMBED), POS)
    o_neg = _tc_out_transpose3(g_neg.reshape(N_NEG // 2, 2 * EMBED), NEG)

    return (o_in.T,
            o_pos.transpose(2, 0, 1),
            o_neg.transpose(2, 0, 1))


def kernel(input_labels, pos_labels, neg_labels, in_table, out_table):
    return _run(input_labels, pos_labels, neg_labels, in_table, out_table)


# speedup vs baseline: 3.2433x; 1.3814x over previous
import functools

import jax
import jax.numpy as jnp
from jax import lax
from jax.experimental import pallas as pl
from jax.experimental.pallas import tpu as pltpu
from jax.experimental.pallas import tpu_sc as plsc

VOCAB = 1000000
EMBED = 64
BATCH = 16384
POS = 10
NEG = 50

NC = 2
NS = 16
NW = NC * NS

CH = 512

N_IN = BATCH
N_POS = BATCH * POS
N_NEG = BATCH * NEG

TL = 8192
BL = 16384

NTB = (VOCAB + TL - 1) // TL
VPAD = NTB * TL


def _tt_body(x_ref, o_ref):
    x = x_ref[...]
    a = x[:, : TL // 2].T
    b = x[:, TL // 2:].T
    o_ref[...] = jnp.concatenate([a, b], axis=1)


def _tc_table_transpose(tab_t):
    return pl.pallas_call(
        _tt_body,
        grid=(NTB,),
        in_specs=[pl.BlockSpec((EMBED, TL), lambda i: (0, i))],
        out_specs=pl.BlockSpec((TL // 2, 2 * EMBED), lambda i: (i, 0)),
        out_shape=jax.ShapeDtypeStruct((VPAD // 2, 2 * EMBED), jnp.float32),
    )(tab_t)


def _to_body(x_ref, o_ref):
    x = x_ref[...]
    a = x[:, :EMBED].T
    b = x[:, EMBED:].T
    o_ref[0, :, :] = jnp.concatenate([a, b], axis=1)


def _tc_out_transpose3(g2, k):
    nb = BATCH // BL
    return pl.pallas_call(
        _to_body,
        grid=(k, nb),
        in_specs=[pl.BlockSpec((BL // 2, 2 * EMBED),
                               lambda i, j: (i * nb + j, 0))],
        out_specs=pl.BlockSpec((1, EMBED, BL), lambda i, j: (i, 0, j)),
        out_shape=jax.ShapeDtypeStruct((k, EMBED, BATCH), jnp.float32),
    )(g2)


def _to2_body(x_ref, o_ref):
    x = x_ref[...]
    a = x[:, :EMBED].T
    b = x[:, EMBED:].T
    o_ref[...] = jnp.concatenate([a, b], axis=1)


def _tc_out_transpose2(g2):
    nb = BATCH // BL
    return pl.pallas_call(
        _to2_body,
        grid=(nb,),
        in_specs=[pl.BlockSpec((BL // 2, 2 * EMBED), lambda j: (j, 0))],
        out_specs=pl.BlockSpec((EMBED, BL), lambda j: (0, j)),
        out_shape=jax.ShapeDtypeStruct((EMBED, BATCH), jnp.float32),
    )(g2)


def _pi(r):
    blk = (r // TL) * TL
    q = r % TL
    return blk + 2 * (q % (TL // 2)) + q // (TL // 2)


def _sigma(lbl_k_major, k):
    x = lbl_k_major.reshape(k, BATCH // BL, 2, BL // 2)
    return x.swapaxes(2, 3).reshape(-1)


def _gather_desc(table, idx_vmem, rows_v, sem, c):
    return pltpu.make_async_copy(
        table.at[idx_vmem.at[pl.ds(c * CH, CH)]], rows_v, sem)


def _store_desc(out_hbm, rows_v, sem, base, c):
    return pltpu.make_async_copy(
        rows_v, out_hbm.at[pl.ds(base + c * CH, CH)], sem)


def _gather_range(table, idx_vmem, out_hbm, rows_a, rows_b,
                  gsa, gsb, ssa, ssb, base, nch):
    if nch == 1:
        _gather_desc(table, idx_vmem, rows_a, gsa, 0).start()
        _gather_desc(table, idx_vmem, rows_a, gsa, 0).wait()
        pltpu.sync_copy(rows_a, out_hbm.at[pl.ds(base, CH)])
        return

    _gather_desc(table, idx_vmem, rows_a, gsa, 0).start()

    def pair(p, carry):
        g = 2 * p
        _gather_desc(table, idx_vmem, rows_b, gsb, g + 1).start()
        _gather_desc(table, idx_vmem, rows_a, gsa, g).wait()
        _store_desc(out_hbm, rows_a, ssa, base, g).start()

        @pl.when(g + 2 < nch)
        def _():
            _store_desc(out_hbm, rows_a, ssa, base, g).wait()
            _gather_desc(table, idx_vmem, rows_a, gsa, g + 2).start()

        _gather_desc(table, idx_vmem, rows_b, gsb, g + 1).wait()
        _store_desc(out_hbm, rows_b, ssb, base, g + 1).start()
        _store_desc(out_hbm, rows_b, ssb, base, g + 1).wait()
        return carry

    lax.fori_loop(0, nch // 2, pair, None)
    _store_desc(out_hbm, rows_a, ssa, base, nch - 2).wait()


def _body(in_table, out_table, in_lbl, pos_lbl, neg_lbl,
          out_in, out_pos, out_neg,
          idx_in_v, idx_pos_v, idx_neg_v, rows_a, rows_b,
          gsa, gsb, ssa, ssb):
    wid = lax.axis_index("s") * NC + lax.axis_index("c")

    in_pw = N_IN // NW
    pos_pw = N_POS // NW
    neg_pw = N_NEG // NW

    pltpu.sync_copy(in_lbl.at[pl.ds(wid * in_pw, in_pw)], idx_in_v)
    pltpu.sync_copy(pos_lbl.at[pl.ds(wid * pos_pw, pos_pw)], idx_pos_v)
    pltpu.sync_copy(neg_lbl.at[pl.ds(wid * neg_pw, neg_pw)], idx_neg_v)

    _gather_range(in_table, idx_in_v, out_in, rows_a, rows_b,
                  gsa, gsb, ssa, ssb, wid * in_pw, in_pw // CH)
    _gather_range(out_table, idx_pos_v, out_pos, rows_a, rows_b,
                  gsa, gsb, ssa, ssb, wid * pos_pw, pos_pw // CH)
    _gather_range(out_table, idx_neg_v, out_neg, rows_a, rows_b,
                  gsa, gsb, ssa, ssb, wid * neg_pw, neg_pw // CH)


def _sc_gather(in_table, out_table, in_lbl, pos_lbl, neg_lbl):
    mesh = plsc.VectorSubcoreMesh(core_axis_name="c", subcore_axis_name="s")
    f = pl.kernel(
        _body,
        out_type=[
            jax.ShapeDtypeStruct((N_IN, EMBED), jnp.float32),
            jax.ShapeDtypeStruct((N_POS, EMBED), jnp.float32),
            jax.ShapeDtypeStruct((N_NEG, EMBED), jnp.float32),
        ],
        mesh=mesh,
        compiler_params=pltpu.CompilerParams(use_tc_tiling_on_sc=False),
        scratch_types=[
            pltpu.VMEM((N_IN // NW,), jnp.int32),
            pltpu.VMEM((N_POS // NW,), jnp.int32),
            pltpu.VMEM((N_NEG // NW,), jnp.int32),
            pltpu.VMEM((CH, EMBED), jnp.float32),
            pltpu.VMEM((CH, EMBED), jnp.float32),
            pltpu.SemaphoreType.DMA,
            pltpu.SemaphoreType.DMA,
            pltpu.SemaphoreType.DMA,
            pltpu.SemaphoreType.DMA,
        ],
    )
    return f(in_table, out_table, in_lbl, pos_lbl, neg_lbl)


@jax.jit
def _run(input_labels, pos_labels, neg_labels, in_table, out_table):
    in_pairs = _tc_table_transpose(in_table.T)
    out_pairs = _tc_table_transpose(out_table.T)
    in_rm = in_pairs.reshape(VPAD, EMBED)
    out_rm = out_pairs.reshape(VPAD, EMBED)

    in_lbl = _sigma(_pi(input_labels.astype(jnp.int32)), 1)
    pos_lbl = _sigma(_pi(pos_labels.astype(jnp.int32).T), POS)
    neg_lbl = _sigma(_pi(neg_labels.astype(jnp.int32).T), NEG)

    g_in, g_pos, g_neg = _sc_gather(in_rm, out_rm, in_lbl, pos_lbl, neg_lbl)

    o_in = _tc_out_transpose2(g_in.reshape(BATCH // 2, 2 * EMBED))
    o_pos = _tc_out_transpose3(g_pos.reshape(N_POS // 2, 2 * EMBED), POS)
    o_neg = _tc_out_transpose3(g_neg.reshape(N_NEG // 2, 2 * EMBED), NEG)

    return (o_in.T,
            o_pos.transpose(2, 0, 1),
            o_neg.transpose(2, 0, 1))


def kernel(input_labels, pos_labels, neg_labels, in_table, out_table):
    return _run(input_labels, pos_labels, neg_labels, in_table, out_table)


# speedup vs baseline: 3.6143x; 1.1144x over previous
import functools

import jax
import jax.numpy as jnp
from jax import lax
from jax.experimental import pallas as pl
from jax.experimental.pallas import tpu as pltpu
from jax.experimental.pallas import tpu_sc as plsc

VOCAB = 1000000
EMBED = 64
BATCH = 16384
POS = 10
NEG = 50

NC = 2
NS = 16
NW = NC * NS

CH = 512

N_IN = BATCH
N_POS = BATCH * POS
N_NEG = BATCH * NEG

TL = 16384
BL = 16384

NTB = (VOCAB + TL - 1) // TL
VPAD = NTB * TL


def _tt_body(x_ref, o_ref):
    x = x_ref[...]
    a = x[:, : TL // 2].T
    b = x[:, TL // 2:].T
    o_ref[...] = jnp.concatenate([a, b], axis=1)


def _tc_table_transpose(tab_t):
    return pl.pallas_call(
        _tt_body,
        grid=(NTB,),
        in_specs=[pl.BlockSpec((EMBED, TL), lambda i: (0, i))],
        out_specs=pl.BlockSpec((TL // 2, 2 * EMBED), lambda i: (i, 0)),
        out_shape=jax.ShapeDtypeStruct((VPAD // 2, 2 * EMBED), jnp.float32),
    )(tab_t)


def _to_body(x_ref, o_ref):
    x = x_ref[...]
    a = x[:, :EMBED].T
    b = x[:, EMBED:].T
    o_ref[0, :, :] = jnp.concatenate([a, b], axis=1)


def _tc_out_transpose3(g2, k):
    nb = BATCH // BL
    return pl.pallas_call(
        _to_body,
        grid=(k, nb),
        in_specs=[pl.BlockSpec((BL // 2, 2 * EMBED),
                               lambda i, j: (i * nb + j, 0))],
        out_specs=pl.BlockSpec((1, EMBED, BL), lambda i, j: (i, 0, j)),
        out_shape=jax.ShapeDtypeStruct((k, EMBED, BATCH), jnp.float32),
    )(g2)


def _to2_body(x_ref, o_ref):
    x = x_ref[...]
    a = x[:, :EMBED].T
    b = x[:, EMBED:].T
    o_ref[...] = jnp.concatenate([a, b], axis=1)


def _tc_out_transpose2(g2):
    nb = BATCH // BL
    return pl.pallas_call(
        _to2_body,
        grid=(nb,),
        in_specs=[pl.BlockSpec((BL // 2, 2 * EMBED), lambda j: (j, 0))],
        out_specs=pl.BlockSpec((EMBED, BL), lambda j: (0, j)),
        out_shape=jax.ShapeDtypeStruct((EMBED, BATCH), jnp.float32),
    )(g2)


def _pi(r):
    blk = (r // TL) * TL
    q = r % TL
    return blk + 2 * (q % (TL // 2)) + q // (TL // 2)


def _sigma(lbl_k_major, k):
    x = lbl_k_major.reshape(k, BATCH // BL, 2, BL // 2)
    return x.swapaxes(2, 3).reshape(-1)


def _gather_desc(table, idx_vmem, rows_v, sem, c):
    return pltpu.make_async_copy(
        table.at[idx_vmem.at[pl.ds(c * CH, CH)]], rows_v, sem)


def _store_desc(out_hbm, rows_v, sem, base, c):
    return pltpu.make_async_copy(
        rows_v, out_hbm.at[pl.ds(base + c * CH, CH)], sem)


def _gather_range(table, idx_vmem, out_hbm, rows_a, rows_b,
                  gsa, gsb, ssa, ssb, base, nch):
    if nch == 1:
        _gather_desc(table, idx_vmem, rows_a, gsa, 0).start()
        _gather_desc(table, idx_vmem, rows_a, gsa, 0).wait()
        pltpu.sync_copy(rows_a, out_hbm.at[pl.ds(base, CH)])
        return

    _gather_desc(table, idx_vmem, rows_a, gsa, 0).start()

    def pair(p, carry):
        g = 2 * p
        _gather_desc(table, idx_vmem, rows_b, gsb, g + 1).start()
        _gather_desc(table, idx_vmem, rows_a, gsa, g).wait()
        _store_desc(out_hbm, rows_a, ssa, base, g).start()

        @pl.when(g + 2 < nch)
        def _():
            _store_desc(out_hbm, rows_a, ssa, base, g).wait()
            _gather_desc(table, idx_vmem, rows_a, gsa, g + 2).start()

        _gather_desc(table, idx_vmem, rows_b, gsb, g + 1).wait()
        _store_desc(out_hbm, rows_b, ssb, base, g + 1).start()
        _store_desc(out_hbm, rows_b, ssb, base, g + 1).wait()
        return carry

    lax.fori_loop(0, nch // 2, pair, None)
    _store_desc(out_hbm, rows_a, ssa, base, nch - 2).wait()


def _body_posneg(out_table, pos_lbl, neg_lbl,
                 out_pos, out_neg,
                 idx_pos_v, idx_neg_v, rows_a, rows_b,
                 gsa, gsb, ssa, ssb):
    wid = lax.axis_index("s") * NC + lax.axis_index("c")

    pos_pw = N_POS // NW
    neg_pw = N_NEG // NW

    pltpu.sync_copy(pos_lbl.at[pl.ds(wid * pos_pw, pos_pw)], idx_pos_v)
    pltpu.sync_copy(neg_lbl.at[pl.ds(wid * neg_pw, neg_pw)], idx_neg_v)

    _gather_range(out_table, idx_pos_v, out_pos, rows_a, rows_b,
                  gsa, gsb, ssa, ssb, wid * pos_pw, pos_pw // CH)
    _gather_range(out_table, idx_neg_v, out_neg, rows_a, rows_b,
                  gsa, gsb, ssa, ssb, wid * neg_pw, neg_pw // CH)


def _body_in(in_table, in_lbl, out_in, idx_in_v, rows_a, rows_b,
             gsa, gsb, ssa, ssb):
    wid = lax.axis_index("s") * NC + lax.axis_index("c")
    in_pw = N_IN // NW
    pltpu.sync_copy(in_lbl.at[pl.ds(wid * in_pw, in_pw)], idx_in_v)
    _gather_range(in_table, idx_in_v, out_in, rows_a, rows_b,
                  gsa, gsb, ssa, ssb, wid * in_pw, in_pw // CH)


def _sc_gather_posneg(out_table, pos_lbl, neg_lbl):
    mesh = plsc.VectorSubcoreMesh(core_axis_name="c", subcore_axis_name="s")
    f = pl.kernel(
        _body_posneg,
        out_type=[
            jax.ShapeDtypeStruct((N_POS, EMBED), jnp.float32),
            jax.ShapeDtypeStruct((N_NEG, EMBED), jnp.float32),
        ],
        mesh=mesh,
        compiler_params=pltpu.CompilerParams(use_tc_tiling_on_sc=False),
        scratch_types=[
            pltpu.VMEM((N_POS // NW,), jnp.int32),
            pltpu.VMEM((N_NEG // NW,), jnp.int32),
            pltpu.VMEM((CH, EMBED), jnp.float32),
            pltpu.VMEM((CH, EMBED), jnp.float32),
            pltpu.SemaphoreType.DMA,
            pltpu.SemaphoreType.DMA,
            pltpu.SemaphoreType.DMA,
            pltpu.SemaphoreType.DMA,
        ],
    )
    return f(out_table, pos_lbl, neg_lbl)


def _sc_gather_in(in_table, in_lbl):
    mesh = plsc.VectorSubcoreMesh(core_axis_name="c", subcore_axis_name="s")
    f = pl.kernel(
        _body_in,
        out_type=[jax.ShapeDtypeStruct((N_IN, EMBED), jnp.float32)],
        mesh=mesh,
        compiler_params=pltpu.CompilerParams(use_tc_tiling_on_sc=False),
        scratch_types=[
            pltpu.VMEM((N_IN // NW,), jnp.int32),
            pltpu.VMEM((CH, EMBED), jnp.float32),
            pltpu.VMEM((CH, EMBED), jnp.float32),
            pltpu.SemaphoreType.DMA,
            pltpu.SemaphoreType.DMA,
            pltpu.SemaphoreType.DMA,
            pltpu.SemaphoreType.DMA,
        ],
    )
    return f(in_table, in_lbl)


@jax.jit
def _run(input_labels, pos_labels, neg_labels, in_table, out_table):
    in_pairs = _tc_table_transpose(in_table.T)
    out_pairs = _tc_table_transpose(out_table.T)
    in_rm = in_pairs.reshape(VPAD, EMBED)
    out_rm = out_pairs.reshape(VPAD, EMBED)

    in_lbl = _sigma(_pi(input_labels.astype(jnp.int32)), 1)
    pos_lbl = _sigma(_pi(pos_labels.astype(jnp.int32).T), POS)
    neg_lbl = _sigma(_pi(neg_labels.astype(jnp.int32).T), NEG)

    g_pos, g_neg = _sc_gather_posneg(out_rm, pos_lbl, neg_lbl)
    (g_in,) = _sc_gather_in(in_rm, in_lbl)

    o_in = _tc_out_transpose2(g_in.reshape(BATCH // 2, 2 * EMBED))
    o_pos = _tc_out_transpose3(g_pos.reshape(N_POS // 2, 2 * EMBED), POS)
    o_neg = _tc_out_transpose3(g_neg.reshape(N_NEG // 2, 2 * EMBED), NEG)

    return (o_in.T,
            o_pos.transpose(2, 0, 1),
            o_neg.transpose(2, 0, 1))


def kernel(input_labels, pos_labels, neg_labels, in_table, out_table):
    return _run(input_labels, pos_labels, neg_labels, in_table, out_table)


# speedup vs baseline: 3.6163x; 1.0006x over previous
import functools

import jax
import jax.numpy as jnp
from jax import lax
from jax.experimental import pallas as pl
from jax.experimental.pallas import tpu as pltpu
from jax.experimental.pallas import tpu_sc as plsc

VOCAB = 1000000
EMBED = 64
BATCH = 16384
POS = 10
NEG = 50

NC = 2
NS = 16
NW = NC * NS

CH = 512

N_IN = BATCH
N_POS = BATCH * POS
N_NEG = BATCH * NEG

TL = 16384
BL = 16384

NTB = (VOCAB + TL - 1) // TL
VPAD = NTB * TL


def _tt_body(x_ref, o_ref):
    x = x_ref[...]
    a = x[:, : TL // 2].T
    b = x[:, TL // 2:].T
    o_ref[...] = jnp.concatenate([a, b], axis=1)


def _tc_table_transpose(tab_t):
    return pl.pallas_call(
        _tt_body,
        grid=(NTB,),
        in_specs=[pl.BlockSpec((EMBED, TL), lambda i: (0, i))],
        out_specs=pl.BlockSpec((TL // 2, 2 * EMBED), lambda i: (i, 0)),
        out_shape=jax.ShapeDtypeStruct((VPAD // 2, 2 * EMBED), jnp.float32),
    )(tab_t)


def _to_body(x_ref, o_ref):
    x = x_ref[...]
    a = x[:, :EMBED].T
    b = x[:, EMBED:].T
    o_ref[0, :, :] = jnp.concatenate([a, b], axis=1)


def _tc_out_transpose3(g2, k):
    nb = BATCH // BL
    return pl.pallas_call(
        _to_body,
        grid=(k, nb),
        in_specs=[pl.BlockSpec((BL // 2, 2 * EMBED),
                               lambda i, j: (i * nb + j, 0))],
        out_specs=pl.BlockSpec((1, EMBED, BL), lambda i, j: (i, 0, j)),
        out_shape=jax.ShapeDtypeStruct((k, EMBED, BATCH), jnp.float32),
    )(g2)


def _to2_body(x_ref, o_ref):
    x = x_ref[...]
    a = x[:, :EMBED].T
    b = x[:, EMBED:].T
    o_ref[...] = jnp.concatenate([a, b], axis=1)


def _tc_out_transpose2(g2):
    nb = BATCH // BL
    return pl.pallas_call(
        _to2_body,
        grid=(nb,),
        in_specs=[pl.BlockSpec((BL // 2, 2 * EMBED), lambda j: (j, 0))],
        out_specs=pl.BlockSpec((EMBED, BL), lambda j: (0, j)),
        out_shape=jax.ShapeDtypeStruct((EMBED, BATCH), jnp.float32),
    )(g2)


def _pi(r):
    blk = (r // TL) * TL
    q = r % TL
    return blk + 2 * (q % (TL // 2)) + q // (TL // 2)


def _sigma(lbl_k_major, k):
    x = lbl_k_major.reshape(k, BATCH // BL, 2, BL // 2)
    return x.swapaxes(2, 3).reshape(-1)


def _gather_desc(table, idx_vmem, rows_v, sem, c):
    return pltpu.make_async_copy(
        table.at[idx_vmem.at[pl.ds(c * CH, CH)]], rows_v, sem)


def _store_desc(out_hbm, rows_v, sem, base, c):
    return pltpu.make_async_copy(
        rows_v, out_hbm.at[pl.ds(base + c * CH, CH)], sem)


def _gather_range(table, idx_vmem, out_hbm, rows_a, rows_b,
                  gsa, gsb, ssa, ssb, base, nch):
    if nch == 1:
        _gather_desc(table, idx_vmem, rows_a, gsa, 0).start()
        _gather_desc(table, idx_vmem, rows_a, gsa, 0).wait()
        pltpu.sync_copy(rows_a, out_hbm.at[pl.ds(base, CH)])
        return

    _gather_desc(table, idx_vmem, rows_a, gsa, 0).start()

    def pair(p, carry):
        g = 2 * p
        _gather_desc(table, idx_vmem, rows_b, gsb, g + 1).start()
        _gather_desc(table, idx_vmem, rows_a, gsa, g).wait()
        _store_desc(out_hbm, rows_a, ssa, base, g).start()

        @pl.when(g + 2 < nch)
        def _():
            _store_desc(out_hbm, rows_a, ssa, base, g).wait()
            _gather_desc(table, idx_vmem, rows_a, gsa, g + 2).start()

        _gather_desc(table, idx_vmem, rows_b, gsb, g + 1).wait()
        _store_desc(out_hbm, rows_b, ssb, base, g + 1).start()
        _store_desc(out_hbm, rows_b, ssb, base, g + 1).wait()
        return carry

    lax.fori_loop(0, nch // 2, pair, None)
    _store_desc(out_hbm, rows_a, ssa, base, nch - 2).wait()


def _body_posneg(out_table, pos_lbl, neg_lbl,
                 out_pos, out_neg,
                 idx_pos_v, idx_neg_v, rows_a, rows_b,
                 gsa, gsb, ssa, ssb):
    wid = lax.axis_index("s") * NC + lax.axis_index("c")

    pos_pw = N_POS // NW
    neg_pw = N_NEG // NW

    pltpu.sync_copy(pos_lbl.at[pl.ds(wid * pos_pw, pos_pw)], idx_pos_v)
    pltpu.sync_copy(neg_lbl.at[pl.ds(wid * neg_pw, neg_pw)], idx_neg_v)

    _gather_range(out_table, idx_pos_v, out_pos, rows_a, rows_b,
                  gsa, gsb, ssa, ssb, wid * pos_pw, pos_pw // CH)
    _gather_range(out_table, idx_neg_v, out_neg, rows_a, rows_b,
                  gsa, gsb, ssa, ssb, wid * neg_pw, neg_pw // CH)


def _body_in(in_table, in_lbl, out_in, idx_in_v, rows_a, rows_b,
             gsa, gsb, ssa, ssb):
    wid = lax.axis_index("s") * NC + lax.axis_index("c")
    in_pw = N_IN // NW
    pltpu.sync_copy(in_lbl.at[pl.ds(wid * in_pw, in_pw)], idx_in_v)
    _gather_range(in_table, idx_in_v, out_in, rows_a, rows_b,
                  gsa, gsb, ssa, ssb, wid * in_pw, in_pw // CH)


def _sc_gather_posneg(out_table, pos_lbl, neg_lbl):
    mesh = plsc.VectorSubcoreMesh(core_axis_name="c", subcore_axis_name="s")
    f = pl.kernel(
        _body_posneg,
        out_type=[
            jax.ShapeDtypeStruct((N_POS, EMBED), jnp.float32),
            jax.ShapeDtypeStruct((N_NEG, EMBED), jnp.float32),
        ],
        mesh=mesh,
        compiler_params=pltpu.CompilerParams(use_tc_tiling_on_sc=False),
        scratch_types=[
            pltpu.VMEM((N_POS // NW,), jnp.int32),
            pltpu.VMEM((N_NEG // NW,), jnp.int32),
            pltpu.VMEM((CH, EMBED), jnp.float32),
            pltpu.VMEM((CH, EMBED), jnp.float32),
            pltpu.SemaphoreType.DMA,
            pltpu.SemaphoreType.DMA,
            pltpu.SemaphoreType.DMA,
            pltpu.SemaphoreType.DMA,
        ],
    )
    return f(out_table, pos_lbl, neg_lbl)


def _sc_gather_in(in_table, in_lbl):
    mesh = plsc.VectorSubcoreMesh(core_axis_name="c", subcore_axis_name="s")
    f = pl.kernel(
        _body_in,
        out_type=[jax.ShapeDtypeStruct((N_IN, EMBED), jnp.float32)],
        mesh=mesh,
        compiler_params=pltpu.CompilerParams(use_tc_tiling_on_sc=False),
        scratch_types=[
            pltpu.VMEM((N_IN // NW,), jnp.int32),
            pltpu.VMEM((CH, EMBED), jnp.float32),
            pltpu.VMEM((CH, EMBED), jnp.float32),
            pltpu.SemaphoreType.DMA,
            pltpu.SemaphoreType.DMA,
            pltpu.SemaphoreType.DMA,
            pltpu.SemaphoreType.DMA,
        ],
    )
    return f(in_table, in_lbl)


@jax.jit
def _run(input_labels, pos_labels, neg_labels, in_table, out_table):
    in_pairs = _tc_table_transpose(in_table.T)
    out_pairs = _tc_table_transpose(out_table.T)
    in_rm = in_pairs.reshape(VPAD, EMBED)
    out_rm = out_pairs.reshape(VPAD, EMBED)

    in_lbl = _sigma(_pi(input_labels.astype(jnp.int32)), 1)
    pos_lbl = _sigma(_pi(pos_labels.astype(jnp.int32).T), POS)
    neg_lbl = _sigma(_pi(neg_labels.astype(jnp.int32).T), NEG)

    (g_in,) = _sc_gather_in(in_rm, in_lbl)
    g_pos, g_neg = _sc_gather_posneg(out_rm, pos_lbl, neg_lbl)

    o_in = _tc_out_transpose2(g_in.reshape(BATCH // 2, 2 * EMBED))
    o_pos = _tc_out_transpose3(g_pos.reshape(N_POS // 2, 2 * EMBED), POS)
    o_neg = _tc_out_transpose3(g_neg.reshape(N_NEG // 2, 2 * EMBED), NEG)

    return (o_in.T,
            o_pos.transpose(2, 0, 1),
            o_neg.transpose(2, 0, 1))


def kernel(input_labels, pos_labels, neg_labels, in_table, out_table):
    return _run(input_labels, pos_labels, neg_labels, in_table, out_table)


# speedup vs baseline: 4.8050x; 1.3287x over previous
import functools

import jax
import jax.numpy as jnp
from jax import lax
from jax.experimental import pallas as pl
from jax.experimental.pallas import tpu as pltpu
from jax.experimental.pallas import tpu_sc as plsc

VOCAB = 1000000
EMBED = 64
BATCH = 16384
POS = 10
NEG = 50

NC = 2
NS = 16
NW = NC * NS

CH = 512

N_IN = BATCH
N_POS = BATCH * POS
N_NEG = BATCH * NEG

TL = 16384
BL = 16384

NTB = (VOCAB + TL - 1) // TL
VPAD = NTB * TL


def _tt_body(x_ref, o_ref):
    x = x_ref[...]
    a = x[:, : TL // 2].T
    b = x[:, TL // 2:].T
    o_ref[...] = jnp.concatenate([a, b], axis=1)


def _tc_table_transpose(tab_t):
    return pl.pallas_call(
        _tt_body,
        grid=(NTB,),
        in_specs=[pl.BlockSpec((EMBED, TL), lambda i: (0, i))],
        out_specs=pl.BlockSpec((TL // 2, 2 * EMBED), lambda i: (i, 0)),
        out_shape=jax.ShapeDtypeStruct((VPAD // 2, 2 * EMBED), jnp.float32),
    )(tab_t)


def _to_body(x_ref, o_ref):
    x = x_ref[...]
    a = x[:, :EMBED].T
    b = x[:, EMBED:].T
    o_ref[0, :, :] = jnp.concatenate([a, b], axis=1)


def _tc_out_transpose3(g2, k):
    nb = BATCH // BL
    return pl.pallas_call(
        _to_body,
        grid=(k, nb),
        in_specs=[pl.BlockSpec((BL // 2, 2 * EMBED),
                               lambda i, j: (i * nb + j, 0))],
        out_specs=pl.BlockSpec((1, EMBED, BL), lambda i, j: (i, 0, j)),
        out_shape=jax.ShapeDtypeStruct((k, EMBED, BATCH), jnp.float32),
    )(g2)


def _to2_body(x_ref, o_ref):
    x = x_ref[...]
    a = x[:, :EMBED].T
    b = x[:, EMBED:].T
    o_ref[...] = jnp.concatenate([a, b], axis=1)


def _tc_out_transpose2(g2):
    nb = BATCH // BL
    return pl.pallas_call(
        _to2_body,
        grid=(nb,),
        in_specs=[pl.BlockSpec((BL // 2, 2 * EMBED), lambda j: (j, 0))],
        out_specs=pl.BlockSpec((EMBED, BL), lambda j: (0, j)),
        out_shape=jax.ShapeDtypeStruct((EMBED, BATCH), jnp.float32),
    )(g2)


def _pi(r):
    blk = (r // TL) * TL
    q = r % TL
    return blk + 2 * (q % (TL // 2)) + q // (TL // 2)


def _sigma(lbl_k_major, k):
    x = lbl_k_major.reshape(k, BATCH // BL, 2, BL // 2)
    return x.swapaxes(2, 3).reshape(-1)


def _gather_desc(table, idx_vmem, rows_v, sem, c):
    return pltpu.make_async_copy(
        table.at[idx_vmem.at[pl.ds(c * CH, CH)]], rows_v, sem)


def _store_desc(out_hbm, rows_v, sem, base, c):
    return pltpu.make_async_copy(
        rows_v, out_hbm.at[pl.ds(base + c * CH, CH)], sem)


def _gather_range(table, idx_vmem, out_hbm, rows_a, rows_b,
                  gsa, gsb, ssa, ssb, base, nch):
    if nch == 1:
        _gather_desc(table, idx_vmem, rows_a, gsa, 0).start()
        _gather_desc(table, idx_vmem, rows_a, gsa, 0).wait()
        pltpu.sync_copy(rows_a, out_hbm.at[pl.ds(base, CH)])
        return

    _gather_desc(table, idx_vmem, rows_a, gsa, 0).start()

    def pair(p, carry):
        g = 2 * p
        _gather_desc(table, idx_vmem, rows_b, gsb, g + 1).start()
        _gather_desc(table, idx_vmem, rows_a, gsa, g).wait()
        _store_desc(out_hbm, rows_a, ssa, base, g).start()

        @pl.when(g + 2 < nch)
        def _():
            _store_desc(out_hbm, rows_a, ssa, base, g).wait()
            _gather_desc(table, idx_vmem, rows_a, gsa, g + 2).start()

        _gather_desc(table, idx_vmem, rows_b, gsb, g + 1).wait()
        _store_desc(out_hbm, rows_b, ssb, base, g + 1).start()
        _store_desc(out_hbm, rows_b, ssb, base, g + 1).wait()
        return carry

    lax.fori_loop(0, nch // 2, pair, None)
    _store_desc(out_hbm, rows_a, ssa, base, nch - 2).wait()


def _pi_vec(r):
    blk = r & ~(TL - 1)
    q = r & (TL - 1)
    lo = q & (TL // 2 - 1)
    hi = q >> ((TL // 2).bit_length() - 1)
    return blk + 2 * lo + hi


def _stage_sigma(lbl, idx_v, half_v, wbase, nch):
    def chunk(c, carry):
        srow = wbase + c * CH
        i = srow // BATCH
        p0 = (srow % BATCH) // 2
        nat = pl.multiple_of(i * BATCH + p0, CH // 2)
        pltpu.sync_copy(lbl.at[pl.ds(nat, CH // 2)],
                        half_v.at[pl.ds(0, CH // 2)])
        pltpu.sync_copy(lbl.at[pl.ds(nat + BATCH // 2, CH // 2)],
                        half_v.at[pl.ds(CH // 2, CH // 2)])

        def vec(s, carry2):
            l = pl.multiple_of(s * 16, 16)
            ii = lax.iota(jnp.int32, 16)
            x = _pi_vec(half_v[pl.ds(l, 16)])
            y = _pi_vec(half_v[pl.ds(CH // 2 + l, 16)])
            dst = c * CH + 2 * (l + ii)
            plsc.store_scatter(idx_v, [dst], x)
            plsc.store_scatter(idx_v, [dst + 1], y)
            return carry2

        lax.fori_loop(0, CH // 32, vec, None)
        return carry

    lax.fori_loop(0, nch, chunk, None)


def _body_posneg(out_table, pos_lbl, neg_lbl,
                 out_pos, out_neg,
                 idx_pos_v, idx_neg_v, half_v, rows_a, rows_b,
                 gsa, gsb, ssa, ssb):
    wid = lax.axis_index("s") * NC + lax.axis_index("c")

    pos_pw = N_POS // NW
    neg_pw = N_NEG // NW

    _stage_sigma(pos_lbl, idx_pos_v, half_v, wid * pos_pw, pos_pw // CH)
    _stage_sigma(neg_lbl, idx_neg_v, half_v, wid * neg_pw, neg_pw // CH)

    _gather_range(out_table, idx_pos_v, out_pos, rows_a, rows_b,
                  gsa, gsb, ssa, ssb, wid * pos_pw, pos_pw // CH)
    _gather_range(out_table, idx_neg_v, out_neg, rows_a, rows_b,
                  gsa, gsb, ssa, ssb, wid * neg_pw, neg_pw // CH)


def _body_in(in_table, in_lbl, out_in, idx_in_v, half_v, rows_a, rows_b,
             gsa, gsb, ssa, ssb):
    wid = lax.axis_index("s") * NC + lax.axis_index("c")
    in_pw = N_IN // NW
    _stage_sigma(in_lbl, idx_in_v, half_v, wid * in_pw, in_pw // CH)
    _gather_range(in_table, idx_in_v, out_in, rows_a, rows_b,
                  gsa, gsb, ssa, ssb, wid * in_pw, in_pw // CH)


def _sc_gather_posneg(out_table, pos_lbl, neg_lbl):
    mesh = plsc.VectorSubcoreMesh(core_axis_name="c", subcore_axis_name="s")
    f = pl.kernel(
        _body_posneg,
        out_type=[
            jax.ShapeDtypeStruct((N_POS, EMBED), jnp.float32),
            jax.ShapeDtypeStruct((N_NEG, EMBED), jnp.float32),
        ],
        mesh=mesh,
        compiler_params=pltpu.CompilerParams(use_tc_tiling_on_sc=False, needs_layout_passes=False),
        scratch_types=[
            pltpu.VMEM((N_POS // NW,), jnp.int32),
            pltpu.VMEM((N_NEG // NW,), jnp.int32),
            pltpu.VMEM((CH,), jnp.int32),
            pltpu.VMEM((CH, EMBED), jnp.float32),
            pltpu.VMEM((CH, EMBED), jnp.float32),
            pltpu.SemaphoreType.DMA,
            pltpu.SemaphoreType.DMA,
            pltpu.SemaphoreType.DMA,
            pltpu.SemaphoreType.DMA,
        ],
    )
    return f(out_table, pos_lbl, neg_lbl)


def _sc_gather_in(in_table, in_lbl):
    mesh = plsc.VectorSubcoreMesh(core_axis_name="c", subcore_axis_name="s")
    f = pl.kernel(
        _body_in,
        out_type=[jax.ShapeDtypeStruct((N_IN, EMBED), jnp.float32)],
        mesh=mesh,
        compiler_params=pltpu.CompilerParams(use_tc_tiling_on_sc=False, needs_layout_passes=False),
        scratch_types=[
            pltpu.VMEM((N_IN // NW,), jnp.int32),
            pltpu.VMEM((CH,), jnp.int32),
            pltpu.VMEM((CH, EMBED), jnp.float32),
            pltpu.VMEM((CH, EMBED), jnp.float32),
            pltpu.SemaphoreType.DMA,
            pltpu.SemaphoreType.DMA,
            pltpu.SemaphoreType.DMA,
            pltpu.SemaphoreType.DMA,
        ],
    )
    return f(in_table, in_lbl)


@jax.jit
def _run(input_labels, pos_labels, neg_labels, in_table, out_table):
    in_pairs = _tc_table_transpose(in_table.T)
    out_pairs = _tc_table_transpose(out_table.T)
    in_rm = in_pairs.reshape(VPAD, EMBED)
    out_rm = out_pairs.reshape(VPAD, EMBED)

    in_lbl = input_labels.astype(jnp.int32)
    pos_lbl = pos_labels.astype(jnp.int32).T.reshape(-1)
    neg_lbl = neg_labels.astype(jnp.int32).T.reshape(-1)

    (g_in,) = _sc_gather_in(in_rm, in_lbl)
    g_pos, g_neg = _sc_gather_posneg(out_rm, pos_lbl, neg_lbl)

    o_in = _tc_out_transpose2(g_in.reshape(BATCH // 2, 2 * EMBED))
    o_pos = _tc_out_transpose3(g_pos.reshape(N_POS // 2, 2 * EMBED), POS)
    o_neg = _tc_out_transpose3(g_neg.reshape(N_NEG // 2, 2 * EMBED), NEG)

    return (o_in.T,
            o_pos.transpose(2, 0, 1),
            o_neg.transpose(2, 0, 1))


def kernel(input_labels, pos_labels, neg_labels, in_table, out_table):
    return _run(input_labels, pos_labels, neg_labels, in_table, out_table)


# speedup vs baseline: 4.9395x; 1.0280x over previous
import functools

import jax
import jax.numpy as jnp
from jax import lax
from jax.experimental import pallas as pl
from jax.experimental.pallas import tpu as pltpu
from jax.experimental.pallas import tpu_sc as plsc

VOCAB = 1000000
EMBED = 64
BATCH = 16384
POS = 10
NEG = 50

NC = 2
NS = 16
NW = NC * NS

CH = 512

N_IN = BATCH
N_POS = BATCH * POS
N_NEG = BATCH * NEG

TL = 16384
BL = 16384

NTB = (VOCAB + TL - 1) // TL
VPAD = NTB * TL


def _tt_body(x_ref, o_ref):
    x = x_ref[...]
    a = x[:, : TL // 2].T
    b = x[:, TL // 2:].T
    o_ref[...] = jnp.concatenate([a, b], axis=1)


def _tc_table_transpose(tab_t):
    return pl.pallas_call(
        _tt_body,
        grid=(NTB,),
        in_specs=[pl.BlockSpec((EMBED, TL), lambda i: (0, i))],
        out_specs=pl.BlockSpec((TL // 2, 2 * EMBED), lambda i: (i, 0)),
        out_shape=jax.ShapeDtypeStruct((VPAD // 2, 2 * EMBED), jnp.float32),
    )(tab_t)


def _to_body(x_ref, o_ref):
    x = x_ref[...]
    a = x[:, :EMBED].T
    b = x[:, EMBED:].T
    o_ref[0, :, :] = jnp.concatenate([a, b], axis=1)


def _tc_out_transpose3(g2, k):
    nb = BATCH // BL
    return pl.pallas_call(
        _to_body,
        grid=(k, nb),
        in_specs=[pl.BlockSpec((BL // 2, 2 * EMBED),
                               lambda i, j: (i * nb + j, 0))],
        out_specs=pl.BlockSpec((1, EMBED, BL), lambda i, j: (i, 0, j)),
        out_shape=jax.ShapeDtypeStruct((k, EMBED, BATCH), jnp.float32),
    )(g2)


def _to2_body(x_ref, o_ref):
    x = x_ref[...]
    a = x[:, :EMBED].T
    b = x[:, EMBED:].T
    o_ref[...] = jnp.concatenate([a, b], axis=1)


def _tc_out_transpose2(g2):
    nb = BATCH // BL
    return pl.pallas_call(
        _to2_body,
        grid=(nb,),
        in_specs=[pl.BlockSpec((BL // 2, 2 * EMBED), lambda j: (j, 0))],
        out_specs=pl.BlockSpec((EMBED, BL), lambda j: (0, j)),
        out_shape=jax.ShapeDtypeStruct((EMBED, BATCH), jnp.float32),
    )(g2)


def _pi(r):
    blk = (r // TL) * TL
    q = r % TL
    return blk + 2 * (q % (TL // 2)) + q // (TL // 2)


def _sigma(lbl_k_major, k):
    x = lbl_k_major.reshape(k, BATCH // BL, 2, BL // 2)
    return x.swapaxes(2, 3).reshape(-1)


def _gather_desc(table, idx_vmem, rows_v, sem, c):
    return pltpu.make_async_copy(
        table.at[idx_vmem.at[pl.ds(c * CH, CH)]], rows_v, sem)


def _store_desc(out_hbm, rows_v, sem, base, c):
    return pltpu.make_async_copy(
        rows_v, out_hbm.at[pl.ds(base + c * CH, CH)], sem)


def _gather_range(table, idx_vmem, out_hbm, rows_a, rows_b,
                  gsa, gsb, ssa, ssb, base, nch):
    if nch == 1:
        _gather_desc(table, idx_vmem, rows_a, gsa, 0).start()
        _gather_desc(table, idx_vmem, rows_a, gsa, 0).wait()
        pltpu.sync_copy(rows_a, out_hbm.at[pl.ds(base, CH)])
        return

    _gather_desc(table, idx_vmem, rows_a, gsa, 0).start()

    def pair(p, carry):
        g = 2 * p
        _gather_desc(table, idx_vmem, rows_b, gsb, g + 1).start()
        _gather_desc(table, idx_vmem, rows_a, gsa, g).wait()
        _store_desc(out_hbm, rows_a, ssa, base, g).start()

        @pl.when(g + 2 < nch)
        def _():
            _store_desc(out_hbm, rows_a, ssa, base, g).wait()
            _gather_desc(table, idx_vmem, rows_a, gsa, g + 2).start()

        _gather_desc(table, idx_vmem, rows_b, gsb, g + 1).wait()
        _store_desc(out_hbm, rows_b, ssb, base, g + 1).start()
        _store_desc(out_hbm, rows_b, ssb, base, g + 1).wait()
        return carry

    lax.fori_loop(0, nch // 2, pair, None)
    _store_desc(out_hbm, rows_a, ssa, base, nch - 2).wait()


def _pi_vec(r):
    blk = r & ~(TL - 1)
    q = r & (TL - 1)
    lo = q & (TL // 2 - 1)
    hi = q >> ((TL // 2).bit_length() - 1)
    return blk + 2 * lo + hi


def _stage_sigma(lbl, idx_v, half_v, half_w, gsem_a, gsem_b, wbase, nch):
    def descs(c, hv, sem):
        srow = wbase + c * CH
        i = srow // BATCH
        p0 = (srow % BATCH) // 2
        nat = pl.multiple_of(i * BATCH + p0, CH // 2)
        return (
            pltpu.make_async_copy(lbl.at[pl.ds(nat, CH // 2)],
                                  hv.at[pl.ds(0, CH // 2)], sem),
            pltpu.make_async_copy(lbl.at[pl.ds(nat + BATCH // 2, CH // 2)],
                                  hv.at[pl.ds(CH // 2, CH // 2)], sem),
        )

    def issue(c, hv, sem):
        d0, d1 = descs(c, hv, sem)
        d0.start()
        d1.start()

    def wait(c, hv, sem):
        d0, d1 = descs(c, hv, sem)
        d0.wait()
        d1.wait()

    def interleave(c, hv):
        def vec(s, carry2):
            l = pl.multiple_of(s * 16, 16)
            ii = lax.iota(jnp.int32, 16)
            x = _pi_vec(hv[pl.ds(l, 16)])
            y = _pi_vec(hv[pl.ds(CH // 2 + l, 16)])
            dst = c * CH + 2 * (l + ii)
            plsc.store_scatter(idx_v, [dst], x)
            plsc.store_scatter(idx_v, [dst + 1], y)
            return carry2

        lax.fori_loop(0, CH // 32, vec, None)

    issue(0, half_v, gsem_a)

    def pair(p, carry):
        c = 2 * p

        @pl.when(c + 1 < nch)
        def _():
            issue(c + 1, half_w, gsem_b)

        wait(c, half_v, gsem_a)
        interleave(c, half_v)

        @pl.when(c + 2 < nch)
        def _():
            issue(c + 2, half_v, gsem_a)

        @pl.when(c + 1 < nch)
        def _():
            wait(c + 1, half_w, gsem_b)
            interleave(c + 1, half_w)

        return carry

    lax.fori_loop(0, (nch + 1) // 2, pair, None)


def _body_posneg(out_table, pos_lbl, neg_lbl,
                 out_pos, out_neg,
                 idx_pos_v, idx_neg_v, half_v, half_w, rows_a, rows_b,
                 gsa, gsb, ssa, ssb):
    wid = lax.axis_index("s") * NC + lax.axis_index("c")

    pos_pw = N_POS // NW
    neg_pw = N_NEG // NW

    _stage_sigma(pos_lbl, idx_pos_v, half_v, half_w, gsa, gsb,
                 wid * pos_pw, pos_pw // CH)
    _stage_sigma(neg_lbl, idx_neg_v, half_v, half_w, gsa, gsb,
                 wid * neg_pw, neg_pw // CH)

    _gather_range(out_table, idx_pos_v, out_pos, rows_a, rows_b,
                  gsa, gsb, ssa, ssb, wid * pos_pw, pos_pw // CH)
    _gather_range(out_table, idx_neg_v, out_neg, rows_a, rows_b,
                  gsa, gsb, ssa, ssb, wid * neg_pw, neg_pw // CH)


def _body_in(in_table, in_lbl, out_in, idx_in_v, half_v, half_w, rows_a,
             rows_b, gsa, gsb, ssa, ssb):
    wid = lax.axis_index("s") * NC + lax.axis_index("c")
    in_pw = N_IN // NW
    _stage_sigma(in_lbl, idx_in_v, half_v, half_w, gsa, gsb,
                 wid * in_pw, in_pw // CH)
    _gather_range(in_table, idx_in_v, out_in, rows_a, rows_b,
                  gsa, gsb, ssa, ssb, wid * in_pw, in_pw // CH)


def _sc_gather_posneg(out_table, pos_lbl, neg_lbl):
    mesh = plsc.VectorSubcoreMesh(core_axis_name="c", subcore_axis_name="s")
    f = pl.kernel(
        _body_posneg,
        out_type=[
            jax.ShapeDtypeStruct((N_POS, EMBED), jnp.float32),
            jax.ShapeDtypeStruct((N_NEG, EMBED), jnp.float32),
        ],
        mesh=mesh,
        compiler_params=pltpu.CompilerParams(use_tc_tiling_on_sc=False, needs_layout_passes=False),
        scratch_types=[
            pltpu.VMEM((N_POS // NW,), jnp.int32),
            pltpu.VMEM((N_NEG // NW,), jnp.int32),
            pltpu.VMEM((CH,), jnp.int32),
            pltpu.VMEM((CH,), jnp.int32),
            pltpu.VMEM((CH, EMBED), jnp.float32),
            pltpu.VMEM((CH, EMBED), jnp.float32),
            pltpu.SemaphoreType.DMA,
            pltpu.SemaphoreType.DMA,
            pltpu.SemaphoreType.DMA,
            pltpu.SemaphoreType.DMA,
        ],
    )
    return f(out_table, pos_lbl, neg_lbl)


def _sc_gather_in(in_table, in_lbl):
    mesh = plsc.VectorSubcoreMesh(core_axis_name="c", subcore_axis_name="s")
    f = pl.kernel(
        _body_in,
        out_type=[jax.ShapeDtypeStruct((N_IN, EMBED), jnp.float32)],
        mesh=mesh,
        compiler_params=pltpu.CompilerParams(use_tc_tiling_on_sc=False, needs_layout_passes=False),
        scratch_types=[
            pltpu.VMEM((N_IN // NW,), jnp.int32),
            pltpu.VMEM((CH,), jnp.int32),
            pltpu.VMEM((CH,), jnp.int32),
            pltpu.VMEM((CH, EMBED), jnp.float32),
            pltpu.VMEM((CH, EMBED), jnp.float32),
            pltpu.SemaphoreType.DMA,
            pltpu.SemaphoreType.DMA,
            pltpu.SemaphoreType.DMA,
            pltpu.SemaphoreType.DMA,
        ],
    )
    return f(in_table, in_lbl)


@jax.jit
def _run(input_labels, pos_labels, neg_labels, in_table, out_table):
    in_pairs = _tc_table_transpose(in_table.T)
    out_pairs = _tc_table_transpose(out_table.T)
    in_rm = in_pairs.reshape(VPAD, EMBED)
    out_rm = out_pairs.reshape(VPAD, EMBED)

    in_lbl = input_labels.astype(jnp.int32)
    pos_lbl = pos_labels.astype(jnp.int32).T.reshape(-1)
    neg_lbl = neg_labels.astype(jnp.int32).T.reshape(-1)

    (g_in,) = _sc_gather_in(in_rm, in_lbl)
    g_pos, g_neg = _sc_gather_posneg(out_rm, pos_lbl, neg_lbl)

    o_in = _tc_out_transpose2(g_in.reshape(BATCH // 2, 2 * EMBED))
    o_pos = _tc_out_transpose3(g_pos.reshape(N_POS // 2, 2 * EMBED), POS)
    o_neg = _tc_out_transpose3(g_neg.reshape(N_NEG // 2, 2 * EMBED), NEG)

    return (o_in.T,
            o_pos.transpose(2, 0, 1),
            o_neg.transpose(2, 0, 1))


def kernel(input_labels, pos_labels, neg_labels, in_table, out_table):
    return _run(input_labels, pos_labels, neg_labels, in_table, out_table)
